# Initial kernel scaffold; baseline (speedup 1.0000x reference)
import functools

import jax
import jax.numpy as jnp
from jax import lax
from jax.experimental import pallas as pl
from jax.experimental.pallas import tpu as pltpu
from jax.experimental.pallas import tpu_sc as plsc

NC = 2
NS = 16
NW = NC * NS
CHUNK = 80


def _worker_id():
    return lax.axis_index("s") * NC + lax.axis_index("c")


@functools.lru_cache(maxsize=None)
def _make_sc_deg(E, N, W16):
    per_w = E // NW
    n_chunks = per_w // CHUNK
    rows_per_sub = N // NS
    mesh = plsc.VectorSubcoreMesh(core_axis_name="c", subcore_axis_name="s")

    @functools.partial(
        pl.kernel, mesh=mesh,
        out_type=jax.ShapeDtypeStruct((NC, N, W16), jnp.float32),
        scratch_types=[
            pltpu.VMEM((CHUNK,), jnp.int32),
            pltpu.VMEM((CHUNK, W16), jnp.float32),
            pltpu.VMEM((rows_per_sub, W16), jnp.float32),
            pltpu.VMEM_SHARED((N, W16), jnp.float32),
            pltpu.SemaphoreType.DMA,
        ],
    )
    def deg_kernel(dst_hbm, ones_hbm, zeros_hbm, out_hbm,
                   idx_v, ones_v, zbuf_v, acc_sh, sem):
        cid = lax.axis_index("c")
        sid = lax.axis_index("s")
        wid = _worker_id()
        pltpu.sync_copy(ones_hbm, ones_v)
        pltpu.sync_copy(zeros_hbm, zbuf_v)
        pltpu.sync_copy(zbuf_v, acc_sh.at[pl.ds(sid * rows_per_sub, rows_per_sub)])
        plsc.subcore_barrier()

        def body(j, _):
            base = wid * per_w + j * CHUNK
            pltpu.sync_copy(dst_hbm.at[pl.ds(base, CHUNK)], idx_v)
            pltpu.sync_copy(ones_v, acc_sh.at[idx_v], add=True)
            return 0

        lax.fori_loop(0, n_chunks, body, 0)
        plsc.subcore_barrier()
        sl = pl.ds(sid * rows_per_sub, rows_per_sub)
        pltpu.sync_copy(acc_sh.at[sl], out_hbm.at[cid, sl])

    return deg_kernel


@functools.lru_cache(maxsize=None)
def _make_sc_edge(E, N, H):
    per_w = E // NW
    n_chunks = per_w // CHUNK
    rows_per_sub = N // NS
    mesh = plsc.VectorSubcoreMesh(core_axis_name="c", subcore_axis_name="s")

    @functools.partial(
        pl.kernel, mesh=mesh,
        out_type=jax.ShapeDtypeStruct((NC, N, H), jnp.float32),
        scratch_types=[
            pltpu.VMEM((CHUNK,), jnp.int32),
            pltpu.VMEM((CHUNK,), jnp.int32),
            pltpu.VMEM((CHUNK, H), jnp.float32),
            pltpu.VMEM((rows_per_sub, H), jnp.float32),
            pltpu.VMEM_SHARED((N, H), jnp.float32),
            pltpu.SemaphoreType.DMA,
        ],
    )
    def edge_kernel(hs_hbm, src_hbm, dst_hbm, zeros_hbm, out_hbm,
                    src_v, dst_v, rows_v, zbuf_v, acc_sh, sem):
        cid = lax.axis_index("c")
        sid = lax.axis_index("s")
        wid = _worker_id()
        pltpu.sync_copy(zeros_hbm, zbuf_v)
        pltpu.sync_copy(zbuf_v, acc_sh.at[pl.ds(sid * rows_per_sub, rows_per_sub)])
        plsc.subcore_barrier()

        def body(j, _):
            base = wid * per_w + j * CHUNK
            pltpu.sync_copy(src_hbm.at[pl.ds(base, CHUNK)], src_v)
            pltpu.sync_copy(dst_hbm.at[pl.ds(base, CHUNK)], dst_v)
            pltpu.async_copy(hs_hbm.at[src_v], rows_v, sem).wait()
            pltpu.sync_copy(rows_v, acc_sh.at[dst_v], add=True)
            return 0

        lax.fori_loop(0, n_chunks, body, 0)
        plsc.subcore_barrier()
        sl = pl.ds(sid * rows_per_sub, rows_per_sub)
        pltpu.sync_copy(acc_sh.at[sl], out_hbm.at[cid, sl])

    return edge_kernel


def _dis_from(degs_ref):
    deg = degs_ref[0, :, 0:1] + degs_ref[1, :, 0:1] + 1.0
    return lax.rsqrt(deg)


def _tc1_body(x_ref, w_ref, degs_ref, hs_ref):
    dis = _dis_from(degs_ref)
    h = jnp.dot(x_ref[...], w_ref[...], preferred_element_type=jnp.float32)
    hs_ref[...] = h * dis


def _tc2_body(hs1_ref, p_ref, degs_ref, w_ref, b1_ref, hs2_ref):
    dis = _dis_from(degs_ref)
    tot = p_ref[0] + p_ref[1] + hs1_ref[...]
    h1 = jnp.maximum(tot * dis + b1_ref[...], 0.0)
    hs2_ref[...] = jnp.dot(h1, w_ref[...], preferred_element_type=jnp.float32) * dis


def _tc3_body(hs2_ref, p_ref, degs_ref, b2_ref, linw_ref, batch_ref, linb_ref,
              out_ref, acc_ref, *, n_blocks, rows, G):
    i = pl.program_id(0)
    dis = _dis_from(degs_ref)
    h2 = (p_ref[0] + p_ref[1] + hs2_ref[...]) * dis + b2_ref[...]
    z = jnp.sum(h2 * linw_ref[...], axis=1)
    b = batch_ref[0, 0, :]
    mask = (lax.broadcasted_iota(jnp.int32, (G, rows), 0)
            == b[None, :]).astype(jnp.float32)
    zc = jnp.concatenate(
        [z[:, None], jnp.ones((rows, 1), jnp.float32),
         jnp.zeros((rows, 6), jnp.float32)], axis=1)
    part = jnp.dot(mask, zc, preferred_element_type=jnp.float32)

    @pl.when(i == 0)
    def _():
        acc_ref[...] = jnp.zeros_like(acc_ref)

    acc_ref[...] += part

    @pl.when(i == n_blocks - 1)
    def _():
        sums = acc_ref[:, 0]
        cnt = acc_ref[:, 1]
        out_ref[0, :] = sums / jnp.maximum(cnt, 1.0) + linb_ref[0, 0]


def kernel(x, edge_index, edge_attr, batch, W1, b1, W2, b2, lin_W, lin_b):
    N, D_IN = x.shape
    E = edge_index.shape[1]
    H = W1.shape[1]
    G = 64
    W16 = 16
    ROWS = 1000
    n_blocks = N // ROWS

    src = edge_index[0]
    dst = edge_index[1]
    rows_per_sub = N // NS
    ones16 = jnp.ones((CHUNK, W16), jnp.float32)
    zeros16 = jnp.zeros((rows_per_sub, W16), jnp.float32)
    zerosH = jnp.zeros((rows_per_sub, H), jnp.float32)

    degs = _make_sc_deg(E, N, W16)(dst, ones16, zeros16)

    hs1 = pl.pallas_call(
        _tc1_body,
        grid=(n_blocks,),
        in_specs=[
            pl.BlockSpec((ROWS, D_IN), lambda i: (i, 0)),
            pl.BlockSpec((D_IN, H), lambda i: (0, 0)),
            pl.BlockSpec((NC, ROWS, W16), lambda i: (0, i, 0)),
        ],
        out_specs=pl.BlockSpec((ROWS, H), lambda i: (i, 0)),
        out_shape=jax.ShapeDtypeStruct((N, H), jnp.float32),
    )(x, W1, degs)

    edge_fn = _make_sc_edge(E, N, H)
    p1 = edge_fn(hs1, src, dst, zerosH)

    hs2 = pl.pallas_call(
        _tc2_body,
        grid=(n_blocks,),
        in_specs=[
            pl.BlockSpec((ROWS, H), lambda i: (i, 0)),
            pl.BlockSpec((NC, ROWS, H), lambda i: (0, i, 0)),
            pl.BlockSpec((NC, ROWS, W16), lambda i: (0, i, 0)),
            pl.BlockSpec((H, H), lambda i: (0, 0)),
            pl.BlockSpec((1, H), lambda i: (0, 0)),
        ],
        out_specs=pl.BlockSpec((ROWS, H), lambda i: (i, 0)),
        out_shape=jax.ShapeDtypeStruct((N, H), jnp.float32),
    )(hs1, p1, degs, W2, b1.reshape(1, H))

    p2 = edge_fn(hs2, src, dst, zerosH)

    out2d = pl.pallas_call(
        functools.partial(_tc3_body, n_blocks=n_blocks, rows=ROWS, G=G),
        grid=(n_blocks,),
        in_specs=[
            pl.BlockSpec((ROWS, H), lambda i: (i, 0)),
            pl.BlockSpec((NC, ROWS, H), lambda i: (0, i, 0)),
            pl.BlockSpec((NC, ROWS, W16), lambda i: (0, i, 0)),
            pl.BlockSpec((1, H), lambda i: (0, 0)),
            pl.BlockSpec((1, H), lambda i: (0, 0)),
            pl.BlockSpec((1, 1, ROWS), lambda i: (i, 0, 0)),
            pl.BlockSpec((1, 1), lambda i: (0, 0)),
        ],
        out_specs=pl.BlockSpec((1, G), lambda i: (0, 0)),
        out_shape=jax.ShapeDtypeStruct((1, G), jnp.float32),
        scratch_shapes=[pltpu.VMEM((G, 8), jnp.float32)],
        compiler_params=pltpu.CompilerParams(
            dimension_semantics=("arbitrary",)),
    )(hs2, p2, degs, b2.reshape(1, H), lin_W.reshape(1, H),
      batch.reshape(n_blocks, 1, ROWS), lin_b.reshape(1, 1))

    return out2d.reshape(G)


# speedup vs baseline: 14.5670x; 14.5670x over previous
import functools

import jax
import jax.numpy as jnp
from jax import lax
from jax.experimental import pallas as pl
from jax.experimental.pallas import tpu as pltpu
from jax.experimental.pallas import tpu_sc as plsc

NC = 2
NS = 16
NW = NC * NS
CHUNK = 80


def _worker_id():
    return lax.axis_index("s") * NC + lax.axis_index("c")


@functools.lru_cache(maxsize=None)
def _make_sc_deg(E, N_pad, W16):
    per_w = E // NW
    n_chunks = per_w // CHUNK
    rows_per_sub = N_pad // NS
    mesh = plsc.VectorSubcoreMesh(core_axis_name="c", subcore_axis_name="s")

    @functools.partial(
        pl.kernel, mesh=mesh,
        out_type=jax.ShapeDtypeStruct((NC, N_pad, W16), jnp.float32),
        scratch_types=[
            pltpu.VMEM((CHUNK,), jnp.int32),
            pltpu.VMEM((CHUNK, W16), jnp.float32),
            pltpu.VMEM((rows_per_sub, W16), jnp.float32),
            pltpu.VMEM_SHARED((N_pad, W16), jnp.float32),
            pltpu.SemaphoreType.DMA,
        ],
        compiler_params=pltpu.CompilerParams(use_tc_tiling_on_sc=False),
    )
    def deg_kernel(dst_hbm, ones_hbm, zeros_hbm, out_hbm,
                   idx_v, ones_v, zbuf_v, acc_sh, sem):
        cid = lax.axis_index("c")
        sid = lax.axis_index("s")
        wid = _worker_id()
        pltpu.sync_copy(ones_hbm, ones_v)
        pltpu.sync_copy(zeros_hbm, zbuf_v)
        pltpu.sync_copy(zbuf_v, acc_sh.at[pl.ds(sid * rows_per_sub, rows_per_sub)])
        plsc.subcore_barrier()

        def body(j, _):
            base = wid * per_w + j * CHUNK
            pltpu.sync_copy(dst_hbm.at[pl.ds(base, CHUNK)], idx_v)
            pltpu.sync_copy(ones_v, acc_sh.at[idx_v], add=True)
            return 0

        lax.fori_loop(0, n_chunks, body, 0)
        plsc.subcore_barrier()
        sl = pl.ds(sid * rows_per_sub, rows_per_sub)
        pltpu.sync_copy(acc_sh.at[sl], out_hbm.at[cid, sl])

    return deg_kernel


@functools.lru_cache(maxsize=None)
def _make_sc_edge(E, N_pad, H):
    per_w = E // NW
    n_chunks = per_w // CHUNK
    rows_per_sub = N_pad // NS
    mesh = plsc.VectorSubcoreMesh(core_axis_name="c", subcore_axis_name="s")

    @functools.partial(
        pl.kernel, mesh=mesh,
        out_type=jax.ShapeDtypeStruct((NC, N_pad, H), jnp.float32),
        scratch_types=[
            pltpu.VMEM((CHUNK,), jnp.int32),
            pltpu.VMEM((CHUNK,), jnp.int32),
            pltpu.VMEM((CHUNK, H), jnp.float32),
            pltpu.VMEM((rows_per_sub, H), jnp.float32),
            pltpu.VMEM_SHARED((N_pad, H), jnp.float32),
            pltpu.SemaphoreType.DMA,
        ],
        compiler_params=pltpu.CompilerParams(use_tc_tiling_on_sc=False),
    )
    def edge_kernel(hs_hbm, src_hbm, dst_hbm, zeros_hbm, out_hbm,
                    src_v, dst_v, rows_v, zbuf_v, acc_sh, sem):
        cid = lax.axis_index("c")
        sid = lax.axis_index("s")
        wid = _worker_id()
        pltpu.sync_copy(zeros_hbm, zbuf_v)
        pltpu.sync_copy(zbuf_v, acc_sh.at[pl.ds(sid * rows_per_sub, rows_per_sub)])
        plsc.subcore_barrier()

        def body(j, _):
            base = wid * per_w + j * CHUNK
            pltpu.sync_copy(src_hbm.at[pl.ds(base, CHUNK)], src_v)
            pltpu.sync_copy(dst_hbm.at[pl.ds(base, CHUNK)], dst_v)
            pltpu.async_copy(hs_hbm.at[src_v], rows_v, sem).wait()
            pltpu.sync_copy(rows_v, acc_sh.at[dst_v], add=True)
            return 0

        lax.fori_loop(0, n_chunks, body, 0)
        plsc.subcore_barrier()
        sl = pl.ds(sid * rows_per_sub, rows_per_sub)
        pltpu.sync_copy(acc_sh.at[sl], out_hbm.at[cid, sl])

    return edge_kernel


def _dis_from(degs_ref):
    deg = degs_ref[0, :, 0:1] + degs_ref[1, :, 0:1] + 1.0
    r = lax.rsqrt(deg)
    return r * (1.5 - 0.5 * deg * r * r)


def _tc1_body(x_ref, w_ref, degs_ref, hs_ref):
    dis = _dis_from(degs_ref)
    h = jnp.dot(x_ref[...], w_ref[...], preferred_element_type=jnp.float32)
    hs_ref[...] = h * dis


def _tc2_body(hs1_ref, p_ref, degs_ref, w_ref, b1_ref, hs2_ref):
    dis = _dis_from(degs_ref)
    tot = p_ref[0] + p_ref[1] + hs1_ref[...]
    h1 = jnp.maximum(tot * dis + b1_ref[...], 0.0)
    hs2_ref[...] = jnp.dot(h1, w_ref[...], preferred_element_type=jnp.float32) * dis


def _tc3_body(hs2_ref, p_ref, degs_ref, b2_ref, linw_ref, batch_ref, linb_ref,
              out_ref, acc_ref, *, n_blocks, rows, G):
    i = pl.program_id(0)
    dis = _dis_from(degs_ref)
    h2 = (p_ref[0] + p_ref[1] + hs2_ref[...]) * dis + b2_ref[...]
    z = jnp.sum(h2 * linw_ref[...], axis=1)
    b = batch_ref[0, 0, :]
    mask = (lax.broadcasted_iota(jnp.int32, (G, rows), 0)
            == b[None, :]).astype(jnp.float32)
    zc = jnp.concatenate(
        [z[:, None], jnp.ones((rows, 1), jnp.float32),
         jnp.zeros((rows, 6), jnp.float32)], axis=1)
    part = jnp.dot(mask, zc, preferred_element_type=jnp.float32)

    @pl.when(i == 0)
    def _():
        acc_ref[...] = jnp.zeros_like(acc_ref)

    acc_ref[...] += part

    @pl.when(i == n_blocks - 1)
    def _():
        sums = acc_ref[:, 0]
        cnt = acc_ref[:, 1]
        out_ref[0, :] = sums / jnp.maximum(cnt, 1.0) + linb_ref[0, 0]


def kernel(x, edge_index, edge_attr, batch, W1, b1, W2, b2, lin_W, lin_b):
    N, D_IN = x.shape
    E = edge_index.shape[1]
    H = W1.shape[1]
    G = 64
    W16 = 16
    ROWS = 1000
    n_blocks = N // ROWS

    src = edge_index[0]
    dst = edge_index[1]
    N_pad = ((N // NS + 7) // 8 * 8) * NS
    rows_per_sub = N_pad // NS
    ones16 = jnp.ones((CHUNK, W16), jnp.float32)
    zeros16 = jnp.zeros((rows_per_sub, W16), jnp.float32)
    zerosH = jnp.zeros((rows_per_sub, H), jnp.float32)

    degs = _make_sc_deg(E, N_pad, W16)(dst, ones16, zeros16)

    hs1 = pl.pallas_call(
        _tc1_body,
        grid=(n_blocks,),
        in_specs=[
            pl.BlockSpec((ROWS, D_IN), lambda i: (i, 0)),
            pl.BlockSpec((D_IN, H), lambda i: (0, 0)),
            pl.BlockSpec((NC, ROWS, W16), lambda i: (0, i, 0)),
        ],
        out_specs=pl.BlockSpec((ROWS, H), lambda i: (i, 0)),
        out_shape=jax.ShapeDtypeStruct((N, H), jnp.float32),
    )(x, W1, degs)

    edge_fn = _make_sc_edge(E, N_pad, H)
    p1 = edge_fn(hs1, src, dst, zerosH)

    hs2 = pl.pallas_call(
        _tc2_body,
        grid=(n_blocks,),
        in_specs=[
            pl.BlockSpec((ROWS, H), lambda i: (i, 0)),
            pl.BlockSpec((NC, ROWS, H), lambda i: (0, i, 0)),
            pl.BlockSpec((NC, ROWS, W16), lambda i: (0, i, 0)),
            pl.BlockSpec((H, H), lambda i: (0, 0)),
            pl.BlockSpec((1, H), lambda i: (0, 0)),
        ],
        out_specs=pl.BlockSpec((ROWS, H), lambda i: (i, 0)),
        out_shape=jax.ShapeDtypeStruct((N, H), jnp.float32),
    )(hs1, p1, degs, W2, b1.reshape(1, H))

    p2 = edge_fn(hs2, src, dst, zerosH)

    out2d = pl.pallas_call(
        functools.partial(_tc3_body, n_blocks=n_blocks, rows=ROWS, G=G),
        grid=(n_blocks,),
        in_specs=[
            pl.BlockSpec((ROWS, H), lambda i: (i, 0)),
            pl.BlockSpec((NC, ROWS, H), lambda i: (0, i, 0)),
            pl.BlockSpec((NC, ROWS, W16), lambda i: (0, i, 0)),
            pl.BlockSpec((1, H), lambda i: (0, 0)),
            pl.BlockSpec((1, H), lambda i: (0, 0)),
            pl.BlockSpec((1, 1, ROWS), lambda i: (i, 0, 0)),
            pl.BlockSpec((1, 1), lambda i: (0, 0)),
        ],
        out_specs=pl.BlockSpec((1, G), lambda i: (0, 0)),
        out_shape=jax.ShapeDtypeStruct((1, G), jnp.float32),
        scratch_shapes=[pltpu.VMEM((G, 8), jnp.float32)],
        compiler_params=pltpu.CompilerParams(
            dimension_semantics=("arbitrary",)),
    )(hs2, p2, degs, b2.reshape(1, H), lin_W.reshape(1, H),
      batch.reshape(n_blocks, 1, ROWS), lin_b.reshape(1, 1))

    return out2d.reshape(G)


# speedup vs baseline: 42.3525x; 2.9074x over previous
import functools

import jax
import jax.numpy as jnp
from jax import lax
from jax.experimental import pallas as pl
from jax.experimental.pallas import tpu as pltpu
from jax.experimental.pallas import tpu_sc as plsc

NC = 2
NS = 16
NW = NC * NS
CHUNK = 80


def _worker_id():
    return lax.axis_index("s") * NC + lax.axis_index("c")


ZROWS = 128


def _zero_acc(zeros_hbm, zbuf_v, acc_sh, sid, rows_per_sub):
    pltpu.sync_copy(zeros_hbm, zbuf_v)
    for k in range(rows_per_sub // ZROWS):
        pltpu.sync_copy(
            zbuf_v, acc_sh.at[pl.ds(sid * rows_per_sub + k * ZROWS, ZROWS)])


@functools.lru_cache(maxsize=None)
def _make_sc_deg(E, N_pad, W16):
    per_w = E // NW
    n_chunks = per_w // CHUNK
    rows_per_sub = N_pad // NS
    mesh = plsc.VectorSubcoreMesh(core_axis_name="c", subcore_axis_name="s")

    @functools.partial(
        pl.kernel, mesh=mesh,
        out_type=jax.ShapeDtypeStruct((NC, N_pad, W16), jnp.float32),
        scratch_types=[
            pltpu.VMEM((n_chunks, CHUNK), jnp.int32),
            pltpu.VMEM((CHUNK, W16), jnp.float32),
            pltpu.VMEM((ZROWS, W16), jnp.float32),
            pltpu.VMEM_SHARED((N_pad, W16), jnp.float32),
            pltpu.SemaphoreType.DMA,
        ],
        compiler_params=pltpu.CompilerParams(use_tc_tiling_on_sc=False),
    )
    def deg_kernel(dst3_hbm, ones_hbm, zeros_hbm, out_hbm,
                   dst_all, ones_v, zbuf_v, acc_sh, sem):
        cid = lax.axis_index("c")
        sid = lax.axis_index("s")
        wid = _worker_id()
        pltpu.sync_copy(dst3_hbm.at[wid], dst_all)
        pltpu.sync_copy(ones_hbm, ones_v)
        _zero_acc(zeros_hbm, zbuf_v, acc_sh, sid, rows_per_sub)
        plsc.subcore_barrier()

        def fire(j, _):
            pltpu.async_copy(ones_v, acc_sh.at[dst_all.at[j]], sem, add=True)
            return 0

        lax.fori_loop(0, n_chunks, fire, 0)

        def drain(j, _):
            pltpu.make_async_copy(ones_v, acc_sh.at[dst_all.at[j]], sem).wait()
            return 0

        lax.fori_loop(0, n_chunks, drain, 0)
        plsc.subcore_barrier()
        sl = pl.ds(sid * rows_per_sub, rows_per_sub)
        pltpu.sync_copy(acc_sh.at[sl], out_hbm.at[cid, sl])

    return deg_kernel


RING = 5
LOOK = 3


@functools.lru_cache(maxsize=None)
def _make_sc_edge(E, N_pad, H):
    per_w = E // NW
    n_chunks = per_w // CHUNK
    rows_per_sub = N_pad // NS
    assert n_chunks % RING == 0
    mesh = plsc.VectorSubcoreMesh(core_axis_name="c", subcore_axis_name="s")

    @functools.partial(
        pl.kernel, mesh=mesh,
        out_type=jax.ShapeDtypeStruct((NC, N_pad, H), jnp.float32),
        scratch_types=[
            pltpu.VMEM((n_chunks, CHUNK), jnp.int32),
            pltpu.VMEM((n_chunks, CHUNK), jnp.int32),
            [pltpu.VMEM((CHUNK, H), jnp.float32)] * RING,
            pltpu.VMEM((ZROWS, H), jnp.float32),
            pltpu.VMEM_SHARED((N_pad, H), jnp.float32),
            [pltpu.SemaphoreType.DMA] * RING,
            [pltpu.SemaphoreType.DMA] * RING,
        ],
        compiler_params=pltpu.CompilerParams(use_tc_tiling_on_sc=False),
    )
    def edge_kernel(hs_hbm, src3_hbm, dst3_hbm, zeros_hbm, out_hbm,
                    src_all, dst_all, rows, zbuf_v, acc_sh, sem_g, sem_s):
        cid = lax.axis_index("c")
        sid = lax.axis_index("s")
        wid = _worker_id()
        pltpu.sync_copy(src3_hbm.at[wid], src_all)
        pltpu.sync_copy(dst3_hbm.at[wid], dst_all)
        _zero_acc(zeros_hbm, zbuf_v, acc_sh, sid, rows_per_sub)
        plsc.subcore_barrier()

        for m in range(LOOK):
            pltpu.async_copy(hs_hbm.at[src_all.at[m]], rows[m], sem_g[m])

        def outer(g, _):
            for b in range(RING):
                m = g * RING + b
                pltpu.make_async_copy(
                    hs_hbm.at[src_all.at[m]], rows[b], sem_g[b]).wait()
                pltpu.async_copy(
                    rows[b], acc_sh.at[dst_all.at[m]], sem_s[b], add=True)
                mf = m + LOOK
                bf = (b + LOOK) % RING

                @pl.when((mf >= RING) & (mf < n_chunks))
                def _():
                    pltpu.make_async_copy(
                        rows[bf], acc_sh.at[dst_all.at[mf - RING]],
                        sem_s[bf]).wait()

                @pl.when(mf < n_chunks)
                def _():
                    pltpu.async_copy(
                        hs_hbm.at[src_all.at[mf]], rows[bf], sem_g[bf])
            return 0

        lax.fori_loop(0, n_chunks // RING, outer, 0)
        for b in range(RING):
            m_last = n_chunks - RING + b
            pltpu.make_async_copy(
                rows[b], acc_sh.at[dst_all.at[m_last]], sem_s[b]).wait()
        plsc.subcore_barrier()
        sl = pl.ds(sid * rows_per_sub, rows_per_sub)
        pltpu.sync_copy(acc_sh.at[sl], out_hbm.at[cid, sl])

    return edge_kernel


def _dis_from(degs_ref):
    deg = degs_ref[0, :, 0:1] + degs_ref[1, :, 0:1] + 1.0
    r = lax.rsqrt(deg)
    return r * (1.5 - 0.5 * deg * r * r)


def _tc1_body(x_ref, w_ref, degs_ref, hs_ref):
    dis = _dis_from(degs_ref)
    h = jnp.dot(x_ref[...], w_ref[...], preferred_element_type=jnp.float32)
    hs_ref[...] = h * dis


def _tc2_body(hs1_ref, p_ref, degs_ref, w_ref, b1_ref, hs2_ref):
    dis = _dis_from(degs_ref)
    tot = p_ref[0] + p_ref[1] + hs1_ref[...]
    h1 = jnp.maximum(tot * dis + b1_ref[...], 0.0)
    hs2_ref[...] = jnp.dot(h1, w_ref[...], preferred_element_type=jnp.float32) * dis


def _tc3_body(hs2_ref, p_ref, degs_ref, b2_ref, linw_ref, batch_ref, linb_ref,
              out_ref, acc_ref, *, n_blocks, rows, G):
    i = pl.program_id(0)
    dis = _dis_from(degs_ref)
    h2 = (p_ref[0] + p_ref[1] + hs2_ref[...]) * dis + b2_ref[...]
    z = jnp.sum(h2 * linw_ref[...], axis=1)
    b = batch_ref[0, 0, :]
    mask = (lax.broadcasted_iota(jnp.int32, (G, rows), 0)
            == b[None, :]).astype(jnp.float32)
    zc = jnp.concatenate(
        [z[:, None], jnp.ones((rows, 1), jnp.float32),
         jnp.zeros((rows, 6), jnp.float32)], axis=1)
    part = jnp.dot(mask, zc, preferred_element_type=jnp.float32)

    @pl.when(i == 0)
    def _():
        acc_ref[...] = jnp.zeros_like(acc_ref)

    acc_ref[...] += part

    @pl.when(i == n_blocks - 1)
    def _():
        sums = acc_ref[:, 0]
        cnt = acc_ref[:, 1]
        out_ref[0, :] = sums / jnp.maximum(cnt, 1.0) + linb_ref[0, 0]


def kernel(x, edge_index, edge_attr, batch, W1, b1, W2, b2, lin_W, lin_b):
    N, D_IN = x.shape
    E = edge_index.shape[1]
    H = W1.shape[1]
    G = 64
    W16 = 16
    ROWS = 1000
    n_blocks = N // ROWS

    n_chunks = (E // NW) // CHUNK
    src3 = edge_index[0].reshape(NW, n_chunks, CHUNK)
    dst3 = edge_index[1].reshape(NW, n_chunks, CHUNK)
    N_pad = ((N // NS + 7) // 8 * 8) * NS
    ones16 = jnp.ones((CHUNK, W16), jnp.float32)
    zeros16 = jnp.zeros((ZROWS, W16), jnp.float32)
    zerosH = jnp.zeros((ZROWS, H), jnp.float32)

    degs = _make_sc_deg(E, N_pad, W16)(dst3, ones16, zeros16)

    hs1 = pl.pallas_call(
        _tc1_body,
        grid=(n_blocks,),
        in_specs=[
            pl.BlockSpec((ROWS, D_IN), lambda i: (i, 0)),
            pl.BlockSpec((D_IN, H), lambda i: (0, 0)),
            pl.BlockSpec((NC, ROWS, W16), lambda i: (0, i, 0)),
        ],
        out_specs=pl.BlockSpec((ROWS, H), lambda i: (i, 0)),
        out_shape=jax.ShapeDtypeStruct((N, H), jnp.float32),
    )(x, W1, degs)

    edge_fn = _make_sc_edge(E, N_pad, H)
    p1 = edge_fn(hs1, src3, dst3, zerosH)

    hs2 = pl.pallas_call(
        _tc2_body,
        grid=(n_blocks,),
        in_specs=[
            pl.BlockSpec((ROWS, H), lambda i: (i, 0)),
            pl.BlockSpec((NC, ROWS, H), lambda i: (0, i, 0)),
            pl.BlockSpec((NC, ROWS, W16), lambda i: (0, i, 0)),
            pl.BlockSpec((H, H), lambda i: (0, 0)),
            pl.BlockSpec((1, H), lambda i: (0, 0)),
        ],
        out_specs=pl.BlockSpec((ROWS, H), lambda i: (i, 0)),
        out_shape=jax.ShapeDtypeStruct((N, H), jnp.float32),
    )(hs1, p1, degs, W2, b1.reshape(1, H))

    p2 = edge_fn(hs2, src3, dst3, zerosH)

    out2d = pl.pallas_call(
        functools.partial(_tc3_body, n_blocks=n_blocks, rows=ROWS, G=G),
        grid=(n_blocks,),
        in_specs=[
            pl.BlockSpec((ROWS, H), lambda i: (i, 0)),
            pl.BlockSpec((NC, ROWS, H), lambda i: (0, i, 0)),
            pl.BlockSpec((NC, ROWS, W16), lambda i: (0, i, 0)),
            pl.BlockSpec((1, H), lambda i: (0, 0)),
            pl.BlockSpec((1, H), lambda i: (0, 0)),
            pl.BlockSpec((1, 1, ROWS), lambda i: (i, 0, 0)),
            pl.BlockSpec((1, 1), lambda i: (0, 0)),
        ],
        out_specs=pl.BlockSpec((1, G), lambda i: (0, 0)),
        out_shape=jax.ShapeDtypeStruct((1, G), jnp.float32),
        scratch_shapes=[pltpu.VMEM((G, 8), jnp.float32)],
        compiler_params=pltpu.CompilerParams(
            dimension_semantics=("arbitrary",)),
    )(hs2, p2, degs, b2.reshape(1, H), lin_W.reshape(1, H),
      batch.reshape(n_blocks, 1, ROWS), lin_b.reshape(1, 1))

    return out2d.reshape(G)
